# baseline (speedup 1.0000x reference)
import functools

import jax
import jax.numpy as jnp
from jax import lax
from jax.experimental import pallas as pl
from jax.experimental.pallas import tpu as pltpu
from jax.experimental.pallas import tpu_sc as plsc

N = 10000
D = 128
E = 320000

NC = 2
NS = 16
L = 128

NP = 10112
RPT = NP // NS
CH0 = 88
CH1 = 70
E0 = NS * CH0 * L
EPAD = NS * (CH0 + CH1) * L
DUMMY = N

_MESH = plsc.VectorSubcoreMesh(
    core_axis_name="c", subcore_axis_name="s", num_cores=NC, num_subcores=NS
)


def _agg_body(x_hbm, src_hbm, dst_hbm, zrow_hbm, zdeg_hbm,
              acc_out, deg_out,
              src_v, dst_v, rows_v, ones_v, degtmp_v, acc_s, deg_s, sem):
    c = lax.axis_index("c")
    s = lax.axis_index("s")
    row0 = s * RPT

    pltpu.sync_copy(src_hbm.at[c, s], src_v)
    pltpu.sync_copy(dst_hbm.at[c, s], dst_v)

    pltpu.sync_copy(zrow_hbm, acc_s.at[pl.ds(row0, RPT)])
    pltpu.sync_copy(zdeg_hbm, degtmp_v)
    pltpu.sync_copy(degtmp_v, deg_s.at[pl.ds(row0, RPT)])

    for i in range(8):
        ones_v[pl.ds(i * 16, 16)] = jnp.full((16,), 1.0, jnp.float32)

    plsc.subcore_barrier()

    def chunk(j, carry):
        pltpu.async_copy(x_hbm.at[src_v.at[j]], rows_v, sem).wait()
        pltpu.sync_copy(rows_v, acc_s.at[dst_v.at[j]], add=True)
        pltpu.sync_copy(ones_v, deg_s.at[dst_v.at[j]], add=True)
        return carry

    nch = jnp.where(c == 0, CH0, CH1)
    lax.fori_loop(0, nch, chunk, 0)

    plsc.subcore_barrier()

    pltpu.sync_copy(acc_s.at[pl.ds(row0, RPT)], acc_out.at[c, pl.ds(row0, RPT)])
    pltpu.sync_copy(deg_s.at[pl.ds(row0, RPT)], degtmp_v)
    pltpu.sync_copy(degtmp_v, deg_out.at[pl.ds(c * NP + row0, RPT)])


_agg = pl.kernel(
    _agg_body,
    out_type=(
        jax.ShapeDtypeStruct((NC, NP, D), jnp.float32),
        jax.ShapeDtypeStruct((NC * NP,), jnp.float32),
    ),
    mesh=_MESH,
    scratch_types=(
        pltpu.VMEM((CH0, L), jnp.int32),
        pltpu.VMEM((CH0, L), jnp.int32),
        pltpu.VMEM((L, D), jnp.float32),
        pltpu.VMEM((L,), jnp.float32),
        pltpu.VMEM((RPT,), jnp.float32),
        pltpu.VMEM_SHARED((NP, D), jnp.float32),
        pltpu.VMEM_SHARED((NP,), jnp.float32),
        pltpu.SemaphoreType.DMA,
    ),
)


def _warm_body(z_hbm, o_hbm, buf_v):
    c = lax.axis_index("c")
    pltpu.sync_copy(z_hbm, buf_v)
    pltpu.sync_copy(buf_v, o_hbm.at[pl.ds(c * RPT, RPT)])


_warm = pl.kernel(
    _warm_body,
    out_type=jax.ShapeDtypeStruct((NC * RPT,), jnp.float32),
    mesh=_MESH,
    scratch_types=(pltpu.VMEM((RPT,), jnp.float32),),
)


def _dense_body(relu, x_ref, acc_ref, deg_ref, ws_ref, bs_ref, wn_ref, bn_ref,
                o_ref):
    deg = jnp.maximum(deg_ref[0] + deg_ref[1], 1.0)
    h_neigh = (acc_ref[0] + acc_ref[1]) / deg[:, None]
    out = (
        jnp.dot(x_ref[...], ws_ref[...], preferred_element_type=jnp.float32)
        + jnp.dot(h_neigh, wn_ref[...], preferred_element_type=jnp.float32)
        + bs_ref[...] + bn_ref[...]
    )
    if relu:
        out = jnp.maximum(out, 0.0)
    o_ref[...] = out


def _dense(x_pad, acc, deg, w_self, b_self, w_neigh, b_neigh, relu):
    return pl.pallas_call(
        functools.partial(_dense_body, relu),
        out_shape=jax.ShapeDtypeStruct((NP, D), jnp.float32),
    )(x_pad, acc, deg, w_self, b_self, w_neigh, b_neigh)


def _split_cores(flat):
    p0 = flat[:E0].reshape(1, NS, CH0, L)
    p1 = flat[E0:].reshape(NS, CH1, L)
    p1 = jnp.concatenate(
        [p1, jnp.zeros((NS, CH0 - CH1, L), jnp.int32)], axis=1
    ).reshape(1, NS, CH0, L)
    return jnp.concatenate([p0, p1], axis=0)


def _prep_edges(edge_index):
    src = jnp.concatenate(
        [edge_index[0], jnp.zeros((EPAD - E,), jnp.int32)])
    dst = jnp.concatenate(
        [edge_index[1], jnp.full((EPAD - E,), DUMMY, jnp.int32)])
    return _split_cores(src), _split_cores(dst)


def kernel(x, edge_index1, edge_index2, W1_self, b1_self, W1_neigh, b1_neigh,
           W2_self, b2_self, W2_neigh, b2_neigh):
    x_pad = jnp.zeros((NP, D), jnp.float32).at[:N].set(x)
    s1, d1 = _prep_edges(edge_index1)
    s2, d2 = _prep_edges(edge_index2)
    zrow = jnp.zeros((RPT, D), jnp.float32)
    zdeg = jnp.zeros((RPT,), jnp.float32)

    acc1, deg1 = _agg(x, s1, d1, zrow, zdeg)
    h = _dense(x_pad, acc1, deg1.reshape(NC, NP), W1_self, b1_self,
               W1_neigh, b1_neigh, True)
    acc2, deg2 = _agg(h, s2, d2, zrow, zdeg)
    out = _dense(h, acc2, deg2.reshape(NC, NP), W2_self, b2_self,
                 W2_neigh, b2_neigh, False)
    return out[:N]


# speedup vs baseline: 1.0781x; 1.0781x over previous
import functools

import jax
import jax.numpy as jnp
from jax import lax
from jax.experimental import pallas as pl
from jax.experimental.pallas import tpu as pltpu
from jax.experimental.pallas import tpu_sc as plsc

N = 10000
D = 128
E = 320000

NC = 2
NS = 16
L = 128

NP = 10112
RPT = NP // NS
CH = 79
EPT = CH * L
EPAD = EPT * NC * NS
DUMMY = N

_MESH = plsc.VectorSubcoreMesh(
    core_axis_name="c", subcore_axis_name="s", num_cores=NC, num_subcores=NS
)


def _agg_body(x_hbm, src_hbm, dst_hbm, zrow_hbm, zdeg_hbm,
              acc_out, deg_out,
              src_v, dst_v, rows_v, ones_v, degtmp_v, acc_s, deg_s, sem,
              sem2):
    c = lax.axis_index("c")
    s = lax.axis_index("s")
    row0 = s * RPT

    pltpu.sync_copy(src_hbm.at[c, s], src_v)
    pltpu.sync_copy(dst_hbm.at[c, s], dst_v)

    pltpu.sync_copy(zrow_hbm, acc_s.at[pl.ds(row0, RPT)])
    pltpu.sync_copy(zdeg_hbm, degtmp_v)
    pltpu.sync_copy(degtmp_v, deg_s.at[pl.ds(row0, RPT)])

    for i in range(8):
        ones_v[pl.ds(i * 16, 16)] = jnp.full((16,), 1.0, jnp.float32)

    plsc.subcore_barrier()

    def chunk(j, carry):
        pltpu.async_copy(x_hbm.at[src_v.at[j]], rows_v, sem).wait()
        pltpu.async_copy(ones_v, deg_s.at[dst_v.at[j]], sem2, add=True)
        pltpu.sync_copy(rows_v, acc_s.at[dst_v.at[j]], add=True)
        return carry

    lax.fori_loop(0, CH, chunk, 0)

    def drain(j, carry):
        pltpu.make_async_copy(ones_v, deg_s.at[dst_v.at[j]], sem2).wait()
        return carry

    lax.fori_loop(0, CH, drain, 0)

    plsc.subcore_barrier()

    pltpu.sync_copy(acc_s.at[pl.ds(row0, RPT)], acc_out.at[c, pl.ds(row0, RPT)])
    pltpu.sync_copy(deg_s.at[pl.ds(row0, RPT)], degtmp_v)
    pltpu.sync_copy(degtmp_v, deg_out.at[pl.ds(c * NP + row0, RPT)])


_agg = pl.kernel(
    _agg_body,
    out_type=(
        jax.ShapeDtypeStruct((NC, NP, D), jnp.float32),
        jax.ShapeDtypeStruct((NC * NP,), jnp.float32),
    ),
    mesh=_MESH,
    scratch_types=(
        pltpu.VMEM((CH, L), jnp.int32),
        pltpu.VMEM((CH, L), jnp.int32),
        pltpu.VMEM((L, D), jnp.float32),
        pltpu.VMEM((L,), jnp.float32),
        pltpu.VMEM((RPT,), jnp.float32),
        pltpu.VMEM_SHARED((NP, D), jnp.float32),
        pltpu.VMEM_SHARED((NP,), jnp.float32),
        pltpu.SemaphoreType.DMA,
        pltpu.SemaphoreType.DMA,
    ),
)


def _warm_body(z_hbm, o_hbm, buf_v):
    c = lax.axis_index("c")
    pltpu.sync_copy(z_hbm, buf_v)
    pltpu.sync_copy(buf_v, o_hbm.at[pl.ds(c * RPT, RPT)])


_warm = pl.kernel(
    _warm_body,
    out_type=jax.ShapeDtypeStruct((NC * RPT,), jnp.float32),
    mesh=_MESH,
    scratch_types=(pltpu.VMEM((RPT,), jnp.float32),),
)


def _dense_body(relu, x_ref, acc_ref, deg_ref, ws_ref, bs_ref, wn_ref, bn_ref,
                o_ref):
    deg = jnp.maximum(deg_ref[0] + deg_ref[1], 1.0)
    h_neigh = (acc_ref[0] + acc_ref[1]) / deg[:, None]
    out = (
        jnp.dot(x_ref[...], ws_ref[...], preferred_element_type=jnp.float32)
        + jnp.dot(h_neigh, wn_ref[...], preferred_element_type=jnp.float32)
        + bs_ref[...] + bn_ref[...]
    )
    if relu:
        out = jnp.maximum(out, 0.0)
    o_ref[...] = out


def _dense(x_pad, acc, deg, w_self, b_self, w_neigh, b_neigh, relu):
    return pl.pallas_call(
        functools.partial(_dense_body, relu),
        out_shape=jax.ShapeDtypeStruct((NP, D), jnp.float32),
    )(x_pad, acc, deg, w_self, b_self, w_neigh, b_neigh)


def _prep_edges(edge_index):
    src = jnp.concatenate(
        [edge_index[0], jnp.zeros((EPAD - E,), jnp.int32)])
    dst = jnp.concatenate(
        [edge_index[1], jnp.full((EPAD - E,), DUMMY, jnp.int32)])
    return src.reshape(NC, NS, CH, L), dst.reshape(NC, NS, CH, L)


def kernel(x, edge_index1, edge_index2, W1_self, b1_self, W1_neigh, b1_neigh,
           W2_self, b2_self, W2_neigh, b2_neigh):
    x_pad = jnp.zeros((NP, D), jnp.float32).at[:N].set(x)
    s1, d1 = _prep_edges(edge_index1)
    s2, d2 = _prep_edges(edge_index2)
    zrow = jnp.zeros((RPT, D), jnp.float32)
    zdeg = jnp.zeros((RPT,), jnp.float32)

    acc1, deg1 = _agg(x, s1, d1, zrow, zdeg)
    h = _dense(x_pad, acc1, deg1.reshape(NC, NP), W1_self, b1_self,
               W1_neigh, b1_neigh, True)
    acc2, deg2 = _agg(h, s2, d2, zrow, zdeg)
    out = _dense(h, acc2, deg2.reshape(NC, NP), W2_self, b2_self,
                 W2_neigh, b2_neigh, False)
    return out[:N]
